# baseline (speedup 1.0000x reference)
import jax
import jax.numpy as jnp
from jax import lax
from jax.experimental import pallas as pl
from jax.experimental.pallas import tpu as pltpu
from jax.experimental.pallas import tpu_sc as plsc

N = 320000
D = 128
S = 10000

NC = 2
NS = 16
NW = NC * NS
CHUNK = N // NW
SP = 10240
SLICE = SP // NS
L = 16

BROWS = 16000
BQ = BROWS // 4
NB = N // BROWS


def _tc_body(x_ref, w_ref, o_ref):
    x = x_ref[...]
    wv = w_ref[...]
    z = lax.dot_general(wv, x, dimension_numbers=(((1,), (1,)), ((), ())),
                        preferred_element_type=jnp.float32)
    o_ref[0] = jnp.exp(z)


NA = 192000
NBLK_A = NA // BROWS
NBLK_B = (N - NA) // BROWS
CH_A = NA // NW
CH_B = (N - NA) // NW


def _make_sums_body(chunk, ids_base):
    def body(y_hbm, ids_hbm, partial_hbm, ids_v, y_v, sums_v,
             spmem_all, tmp_v, acc_v, sem1, sem2):
        c = lax.axis_index("c")
        s = lax.axis_index("s")
        w = c * NS + s
        base = w * chunk
        d1 = pltpu.async_copy(
            ids_hbm.at[pl.ds(ids_base + base, chunk)], ids_v, sem1)
        d2 = pltpu.async_copy(y_hbm.at[pl.ds(base, chunk)], y_v, sem2)

        @plsc.parallel_loop(0, SP // L, unroll=4)
        def _zero(i):
            off = pl.multiple_of(i * L, L)
            sums_v[pl.ds(off, L)] = jnp.zeros((L,), jnp.float32)

        d1.wait()
        d2.wait()
        UN = 5

        def scat_body(i, carry):
            base_off = pl.multiple_of(i * (L * UN), L)
            for k in range(UN):
                off = base_off + k * L
                ids16 = ids_v[pl.ds(off, L)]
                y16 = y_v[pl.ds(off, L)]
                plsc.addupdate_scatter(sums_v, [ids16], y16)
            return carry
        lax.fori_loop(0, chunk // (L * UN), scat_body, 0)

        pltpu.sync_copy(sums_v, spmem_all.at[s])
        plsc.subcore_barrier()

        seg0 = pl.multiple_of(s * SLICE, L)
        pltpu.sync_copy(spmem_all.at[:, pl.ds(seg0, SLICE)], tmp_v)

        @plsc.parallel_loop(0, SLICE // L, unroll=2)
        def _combine(i):
            off = pl.multiple_of(i * L, L)
            acc = tmp_v[0, pl.ds(off, L)]
            for t in range(1, NS):
                acc = acc + tmp_v[t, pl.ds(off, L)]
            acc_v[pl.ds(off, L)] = acc

        pltpu.sync_copy(acc_v, partial_hbm.at[c, pl.ds(seg0, SLICE)])
    return body


CH_B2 = (N - NA) // NS
W_SPLIT = NA // CHUNK
REM_A = NA - W_SPLIT * CHUNK


def _fused_b_body(ya_hbm, yb_hbm, ids_hbm, pa_hbm, out_hbm,
                  ids_v, y_v, sums_v, spmem_all, spmem_final, tmp_v, acc_v,
                  pa_v, p0_v, nids_v, ny_v, out_v,
                  sem1, sem2, sem3, sem4):
    c = lax.axis_index("c")
    s = lax.axis_index("s")
    w = c * NS + s
    sbase = s * CH_B2
    nbase = w * CHUNK
    d1 = pltpu.async_copy(ids_hbm.at[pl.ds(NA + sbase, CH_B2)], ids_v, sem1)
    d2 = pltpu.async_copy(yb_hbm.at[pl.ds(sbase, CH_B2)], y_v, sem2)
    d3 = pltpu.async_copy(ids_hbm.at[pl.ds(nbase, CHUNK)], nids_v, sem3)
    d4 = pltpu.async_copy(pa_hbm, pa_v, sem4)

    @plsc.parallel_loop(0, SP // L, unroll=4)
    def _zero(i):
        off = pl.multiple_of(i * L, L)
        sums_v[pl.ds(off, L)] = jnp.zeros((L,), jnp.float32)

    d1.wait()
    d2.wait()
    UN = 5

    def scat_body(i, carry):
        base_off = pl.multiple_of(i * (L * UN), L)
        for k in range(UN):
            off = base_off + k * L
            ids16 = ids_v[pl.ds(off, L)]
            y16 = y_v[pl.ds(off, L)]
            plsc.addupdate_scatter(sums_v, [ids16], y16)
        return carry
    lax.fori_loop(0, CH_B2 // (L * UN), scat_body, 0)

    pltpu.sync_copy(sums_v, spmem_all.at[s])
    plsc.subcore_barrier()

    seg0 = pl.multiple_of(s * SLICE, L)
    pltpu.sync_copy(spmem_all.at[:, pl.ds(seg0, SLICE)], tmp_v)

    @plsc.parallel_loop(0, SLICE // L, unroll=2)
    def _combine(i):
        off = pl.multiple_of(i * L, L)
        acc = tmp_v[0, pl.ds(off, L)]
        for t in range(1, NS):
            acc = acc + tmp_v[t, pl.ds(off, L)]
        acc_v[pl.ds(off, L)] = acc

    pltpu.sync_copy(acc_v, spmem_final.at[pl.ds(seg0, SLICE)])
    plsc.subcore_barrier()
    pltpu.sync_copy(spmem_final, sums_v)

    d4.wait()

    @plsc.parallel_loop(0, SP // L, unroll=4)
    def _padd(i):
        off = pl.multiple_of(i * L, L)
        p0_v[pl.ds(off, L)] = (
            (pa_v[0, pl.ds(off, L)] + pa_v[1, pl.ds(off, L)])
            + sums_v[pl.ds(off, L)])

    d3.wait()

    @pl.when(w < W_SPLIT)
    def _():
        pltpu.sync_copy(ya_hbm.at[pl.ds(nbase, CHUNK)], ny_v)

    @pl.when(w == W_SPLIT)
    def _():
        pltpu.sync_copy(ya_hbm.at[pl.ds(W_SPLIT * CHUNK, REM_A)],
                        ny_v.at[pl.ds(0, REM_A)])
        pltpu.sync_copy(yb_hbm.at[pl.ds(0, CHUNK - REM_A)],
                        ny_v.at[pl.ds(REM_A, CHUNK - REM_A)])

    @pl.when(w > W_SPLIT)
    def _():
        pltpu.sync_copy(yb_hbm.at[pl.ds(nbase - NA, CHUNK)], ny_v)

    @plsc.parallel_loop(0, CHUNK // L, unroll=4)
    def _norm(i):
        off = pl.multiple_of(i * L, L)
        ids16 = nids_v[pl.ds(off, L)]
        denom = plsc.load_gather(p0_v, [ids16])
        out_v[pl.ds(off, L)] = ny_v[pl.ds(off, L)] / denom

    pltpu.sync_copy(out_v, out_hbm.at[pl.ds(nbase, CHUNK)])


_SC_KERNELS = None


def _sc_kernels():
    global _SC_KERNELS
    if _SC_KERNELS is None:
        mesh = plsc.VectorSubcoreMesh(
            core_axis_name="c", subcore_axis_name="s",
            num_cores=NC, num_subcores=NS)
        sc_params = pltpu.CompilerParams(needs_layout_passes=False)

        def make_sums(chunk, ids_base):
            return pl.kernel(
                _make_sums_body(chunk, ids_base),
                out_type=jax.ShapeDtypeStruct((NC, SP), jnp.float32),
                mesh=mesh,
                compiler_params=sc_params,
                scratch_types=[
                    pltpu.VMEM((chunk,), jnp.int32),
                    pltpu.VMEM((chunk,), jnp.float32),
                    pltpu.VMEM((SP,), jnp.float32),
                    pltpu.VMEM_SHARED((NS, SP), jnp.float32),
                    pltpu.VMEM((NS, SLICE), jnp.float32),
                    pltpu.VMEM((SLICE,), jnp.float32),
                    pltpu.SemaphoreType.DMA,
                    pltpu.SemaphoreType.DMA,
                ],
            )
        sums_a = make_sums(CH_A, 0)
        fused_b = pl.kernel(
            _fused_b_body,
            out_type=jax.ShapeDtypeStruct((N,), jnp.float32),
            mesh=mesh,
            compiler_params=sc_params,
            scratch_types=[
                pltpu.VMEM((CH_B2,), jnp.int32),
                pltpu.VMEM((CH_B2,), jnp.float32),
                pltpu.VMEM((SP,), jnp.float32),
                pltpu.VMEM_SHARED((NS, SP), jnp.float32),
                pltpu.VMEM_SHARED((SP,), jnp.float32),
                pltpu.VMEM((NS, SLICE), jnp.float32),
                pltpu.VMEM((SLICE,), jnp.float32),
                pltpu.VMEM((NC, SP), jnp.float32),
                pltpu.VMEM((SP,), jnp.float32),
                pltpu.VMEM((CHUNK,), jnp.int32),
                pltpu.VMEM((CHUNK,), jnp.float32),
                pltpu.VMEM((CHUNK,), jnp.float32),
                pltpu.SemaphoreType.DMA,
                pltpu.SemaphoreType.DMA,
                pltpu.SemaphoreType.DMA,
                pltpu.SemaphoreType.DMA,
            ],
        )
        _SC_KERNELS = (sums_a, fused_b)
    return _SC_KERNELS

def _make_tc_call(nblocks, block_off):
    return pl.pallas_call(
        _tc_body,
        grid=(nblocks,),
        in_specs=[
            pl.BlockSpec((BROWS, D), lambda i: (i + block_off, 0)),
            pl.BlockSpec((1, D), lambda i: (0, 0)),
        ],
        out_specs=pl.BlockSpec((1, 1, BROWS), lambda i: (i, 0, 0)),
        out_shape=jax.ShapeDtypeStruct((nblocks, 1, BROWS), jnp.float32),
        compiler_params=pltpu.CompilerParams(
            dimension_semantics=("parallel",)),
    )


_tc_call_a = _make_tc_call(NBLK_A, 0)
_tc_call_b = _make_tc_call(NBLK_B, NBLK_A)


def kernel(X, segment_ids, W, b):
    del b
    sums_a, fused_b = _sc_kernels()
    ids = segment_ids.astype(jnp.int32)
    ya = _tc_call_a(X, W).reshape(NA)
    yb = _tc_call_b(X, W).reshape(N - NA)
    pa = sums_a(ya, ids)
    out = fused_b(ya, yb, ids, pa)
    return out.reshape(N, 1)


# speedup vs baseline: 1.0559x; 1.0559x over previous
import jax
import jax.numpy as jnp
from jax import lax
from jax.experimental import pallas as pl
from jax.experimental.pallas import tpu as pltpu
from jax.experimental.pallas import tpu_sc as plsc

N = 320000
D = 128
S = 10000

NC = 2
NS = 16
NW = NC * NS
CHUNK = N // NW
SP = 10240
SLICE = SP // NS
L = 16

BROWS = 16000
BQ = BROWS // 4
NB = N // BROWS


def _tc_body(x_ref, w_ref, o_ref):
    x = x_ref[...]
    wv = w_ref[...]
    z = lax.dot_general(wv, x, dimension_numbers=(((1,), (1,)), ((), ())),
                        preferred_element_type=jnp.float32)
    o_ref[0] = jnp.exp(z)


NA = 256000
NBLK_A = NA // BROWS
NBLK_B = (N - NA) // BROWS
CH_A = NA // NW
CH_B = (N - NA) // NW


def _make_sums_body(chunk, ids_base):
    def body(y_hbm, ids_hbm, partial_hbm, ids_v, y_v, sums_v,
             spmem_all, tmp_v, acc_v, sem1, sem2):
        c = lax.axis_index("c")
        s = lax.axis_index("s")
        w = c * NS + s
        base = w * chunk
        d1 = pltpu.async_copy(
            ids_hbm.at[pl.ds(ids_base + base, chunk)], ids_v, sem1)
        d2 = pltpu.async_copy(y_hbm.at[pl.ds(base, chunk)], y_v, sem2)

        @plsc.parallel_loop(0, SP // L, unroll=4)
        def _zero(i):
            off = pl.multiple_of(i * L, L)
            sums_v[pl.ds(off, L)] = jnp.zeros((L,), jnp.float32)

        d1.wait()
        d2.wait()
        UN = 5

        def scat_body(i, carry):
            base_off = pl.multiple_of(i * (L * UN), L)
            for k in range(UN):
                off = base_off + k * L
                ids16 = ids_v[pl.ds(off, L)]
                y16 = y_v[pl.ds(off, L)]
                plsc.addupdate_scatter(sums_v, [ids16], y16)
            return carry
        lax.fori_loop(0, chunk // (L * UN), scat_body, 0)

        pltpu.sync_copy(sums_v, spmem_all.at[s])
        plsc.subcore_barrier()

        seg0 = pl.multiple_of(s * SLICE, L)
        pltpu.sync_copy(spmem_all.at[:, pl.ds(seg0, SLICE)], tmp_v)

        @plsc.parallel_loop(0, SLICE // L, unroll=2)
        def _combine(i):
            off = pl.multiple_of(i * L, L)
            acc = tmp_v[0, pl.ds(off, L)]
            for t in range(1, NS):
                acc = acc + tmp_v[t, pl.ds(off, L)]
            acc_v[pl.ds(off, L)] = acc

        pltpu.sync_copy(acc_v, partial_hbm.at[c, pl.ds(seg0, SLICE)])
    return body


CH_B2 = (N - NA) // NS
W_SPLIT = NA // CHUNK
REM_A = NA - W_SPLIT * CHUNK


def _fused_b_body(ya_hbm, yb_hbm, ids_hbm, pa_hbm, out_hbm,
                  ids_v, y_v, sums_v, spmem_all, spmem_final, tmp_v, acc_v,
                  pa_v, p0_v, nids_v, ny_v, out_v,
                  sem1, sem2, sem3, sem4):
    c = lax.axis_index("c")
    s = lax.axis_index("s")
    w = c * NS + s
    sbase = s * CH_B2
    nbase = w * CHUNK
    d1 = pltpu.async_copy(ids_hbm.at[pl.ds(NA + sbase, CH_B2)], ids_v, sem1)
    d2 = pltpu.async_copy(yb_hbm.at[pl.ds(sbase, CH_B2)], y_v, sem2)
    d3 = pltpu.async_copy(ids_hbm.at[pl.ds(nbase, CHUNK)], nids_v, sem3)
    d4 = pltpu.async_copy(pa_hbm, pa_v, sem4)

    @plsc.parallel_loop(0, SP // L, unroll=4)
    def _zero(i):
        off = pl.multiple_of(i * L, L)
        sums_v[pl.ds(off, L)] = jnp.zeros((L,), jnp.float32)

    d1.wait()
    d2.wait()
    UN = 5

    def scat_body(i, carry):
        base_off = pl.multiple_of(i * (L * UN), L)
        for k in range(UN):
            off = base_off + k * L
            ids16 = ids_v[pl.ds(off, L)]
            y16 = y_v[pl.ds(off, L)]
            plsc.addupdate_scatter(sums_v, [ids16], y16)
        return carry
    lax.fori_loop(0, CH_B2 // (L * UN), scat_body, 0)

    pltpu.sync_copy(sums_v, spmem_all.at[s])
    plsc.subcore_barrier()

    seg0 = pl.multiple_of(s * SLICE, L)
    pltpu.sync_copy(spmem_all.at[:, pl.ds(seg0, SLICE)], tmp_v)

    @plsc.parallel_loop(0, SLICE // L, unroll=2)
    def _combine(i):
        off = pl.multiple_of(i * L, L)
        acc = tmp_v[0, pl.ds(off, L)]
        for t in range(1, NS):
            acc = acc + tmp_v[t, pl.ds(off, L)]
        acc_v[pl.ds(off, L)] = acc

    pltpu.sync_copy(acc_v, spmem_final.at[pl.ds(seg0, SLICE)])
    plsc.subcore_barrier()
    pltpu.sync_copy(spmem_final, sums_v)

    d4.wait()

    @plsc.parallel_loop(0, SP // L, unroll=4)
    def _padd(i):
        off = pl.multiple_of(i * L, L)
        p0_v[pl.ds(off, L)] = (
            (pa_v[0, pl.ds(off, L)] + pa_v[1, pl.ds(off, L)])
            + sums_v[pl.ds(off, L)])

    d3.wait()

    @pl.when(w < W_SPLIT)
    def _():
        pltpu.sync_copy(ya_hbm.at[pl.ds(nbase, CHUNK)], ny_v)

    @pl.when(w == W_SPLIT)
    def _():
        pltpu.sync_copy(ya_hbm.at[pl.ds(W_SPLIT * CHUNK, REM_A)],
                        ny_v.at[pl.ds(0, REM_A)])
        pltpu.sync_copy(yb_hbm.at[pl.ds(0, CHUNK - REM_A)],
                        ny_v.at[pl.ds(REM_A, CHUNK - REM_A)])

    @pl.when(w > W_SPLIT)
    def _():
        pltpu.sync_copy(yb_hbm.at[pl.ds(nbase - NA, CHUNK)], ny_v)

    @plsc.parallel_loop(0, CHUNK // L, unroll=4)
    def _norm(i):
        off = pl.multiple_of(i * L, L)
        ids16 = nids_v[pl.ds(off, L)]
        denom = plsc.load_gather(p0_v, [ids16])
        out_v[pl.ds(off, L)] = ny_v[pl.ds(off, L)] / denom

    pltpu.sync_copy(out_v, out_hbm.at[pl.ds(nbase, CHUNK)])


_SC_KERNELS = None


def _sc_kernels():
    global _SC_KERNELS
    if _SC_KERNELS is None:
        mesh = plsc.VectorSubcoreMesh(
            core_axis_name="c", subcore_axis_name="s",
            num_cores=NC, num_subcores=NS)
        sc_params = pltpu.CompilerParams(needs_layout_passes=False)

        def make_sums(chunk, ids_base):
            return pl.kernel(
                _make_sums_body(chunk, ids_base),
                out_type=jax.ShapeDtypeStruct((NC, SP), jnp.float32),
                mesh=mesh,
                compiler_params=sc_params,
                scratch_types=[
                    pltpu.VMEM((chunk,), jnp.int32),
                    pltpu.VMEM((chunk,), jnp.float32),
                    pltpu.VMEM((SP,), jnp.float32),
                    pltpu.VMEM_SHARED((NS, SP), jnp.float32),
                    pltpu.VMEM((NS, SLICE), jnp.float32),
                    pltpu.VMEM((SLICE,), jnp.float32),
                    pltpu.SemaphoreType.DMA,
                    pltpu.SemaphoreType.DMA,
                ],
            )
        sums_a = make_sums(CH_A, 0)
        fused_b = pl.kernel(
            _fused_b_body,
            out_type=jax.ShapeDtypeStruct((N,), jnp.float32),
            mesh=mesh,
            compiler_params=sc_params,
            scratch_types=[
                pltpu.VMEM((CH_B2,), jnp.int32),
                pltpu.VMEM((CH_B2,), jnp.float32),
                pltpu.VMEM((SP,), jnp.float32),
                pltpu.VMEM_SHARED((NS, SP), jnp.float32),
                pltpu.VMEM_SHARED((SP,), jnp.float32),
                pltpu.VMEM((NS, SLICE), jnp.float32),
                pltpu.VMEM((SLICE,), jnp.float32),
                pltpu.VMEM((NC, SP), jnp.float32),
                pltpu.VMEM((SP,), jnp.float32),
                pltpu.VMEM((CHUNK,), jnp.int32),
                pltpu.VMEM((CHUNK,), jnp.float32),
                pltpu.VMEM((CHUNK,), jnp.float32),
                pltpu.SemaphoreType.DMA,
                pltpu.SemaphoreType.DMA,
                pltpu.SemaphoreType.DMA,
                pltpu.SemaphoreType.DMA,
            ],
        )
        _SC_KERNELS = (sums_a, fused_b)
    return _SC_KERNELS

def _make_tc_call(nblocks, block_off):
    return pl.pallas_call(
        _tc_body,
        grid=(nblocks,),
        in_specs=[
            pl.BlockSpec((BROWS, D), lambda i: (i + block_off, 0)),
            pl.BlockSpec((1, D), lambda i: (0, 0)),
        ],
        out_specs=pl.BlockSpec((1, 1, BROWS), lambda i: (i, 0, 0)),
        out_shape=jax.ShapeDtypeStruct((nblocks, 1, BROWS), jnp.float32),
        compiler_params=pltpu.CompilerParams(
            dimension_semantics=("parallel",)),
    )


_tc_call_a = _make_tc_call(NBLK_A, 0)
_tc_call_b = _make_tc_call(NBLK_B, NBLK_A)


def kernel(X, segment_ids, W, b):
    del b
    sums_a, fused_b = _sc_kernels()
    ids = segment_ids.astype(jnp.int32)
    ya = _tc_call_a(X, W).reshape(NA)
    yb = _tc_call_b(X, W).reshape(N - NA)
    pa = sums_a(ya, ids)
    out = fused_b(ya, yb, ids, pa)
    return out.reshape(N, 1)
